# Initial kernel scaffold; baseline (speedup 1.0000x reference)
import jax
import jax.numpy as jnp
from jax.experimental import pallas as pl


def kernel(x, edge_index, W1_l, b1, W1_r, W2_l, b2, W2_r):
    raise NotImplementedError("write your pallas kernel here")


# speedup vs baseline: 6.9279x; 6.9279x over previous
import functools

import jax
import jax.numpy as jnp
from jax import lax
from jax.experimental import pallas as pl
from jax.experimental.pallas import tpu as pltpu
from jax.experimental.pallas import tpu_sc as plsc

N_NODES = 10000
N_EDGES = 320000
D_IN = 128
D_H = 32

NC, NS = 2, 16
NW = NC * NS
K = 80
CHUNKS = N_EDGES // (NW * K)
RPS = N_NODES // NS

ROW_BLK = 2000


def _project_body(x_ref, wl_ref, wr_ref, p_ref, r_ref):
    xb = x_ref[...]
    p_ref[...] = jnp.dot(xb, wl_ref[...], preferred_element_type=jnp.float32)
    r_ref[...] = jnp.dot(xb, wr_ref[...], preferred_element_type=jnp.float32)


def _project(x, w_l, w_r):
    n, d = x.shape
    return pl.pallas_call(
        _project_body,
        grid=(n // ROW_BLK,),
        in_specs=[
            pl.BlockSpec((ROW_BLK, d), lambda i: (i, 0)),
            pl.BlockSpec((d, D_H), lambda i: (0, 0)),
            pl.BlockSpec((d, D_H), lambda i: (0, 0)),
        ],
        out_specs=[
            pl.BlockSpec((ROW_BLK, D_H), lambda i: (i, 0)),
            pl.BlockSpec((ROW_BLK, D_H), lambda i: (i, 0)),
        ],
        out_shape=[jax.ShapeDtypeStruct((n, D_H), jnp.float32)] * 2,
    )(x, w_l, w_r)


def _combine1_body(agg_ref, degp_ref, b_ref, r_ref, wl_ref, wr_ref,
                   p2_ref, r2_ref, rdeg_ref):
    ones = jnp.ones((NW, 1), jnp.float32)
    deg = lax.dot_general(degp_ref[0], ones, (((0,), (0,)), ((), ())),
                          preferred_element_type=jnp.float32)
    rdeg = 1.0 / jnp.maximum(deg, 1.0)
    h = jnp.maximum(
        (agg_ref[0] + agg_ref[1]) * rdeg + b_ref[...] + r_ref[...], 0.0)
    p2_ref[...] = jnp.dot(h, wl_ref[...], preferred_element_type=jnp.float32)
    r2_ref[...] = jnp.dot(h, wr_ref[...], preferred_element_type=jnp.float32)
    rdeg_ref[...] = rdeg


def _combine1(agg, degp, b, r, w_l, w_r):
    return pl.pallas_call(
        _combine1_body,
        grid=(N_NODES // ROW_BLK,),
        in_specs=[
            pl.BlockSpec((NC, ROW_BLK, D_H), lambda i: (0, i, 0)),
            pl.BlockSpec((1, NW, ROW_BLK), lambda i: (i, 0, 0)),
            pl.BlockSpec((1, D_H), lambda i: (0, 0)),
            pl.BlockSpec((ROW_BLK, D_H), lambda i: (i, 0)),
            pl.BlockSpec((D_H, D_H), lambda i: (0, 0)),
            pl.BlockSpec((D_H, D_H), lambda i: (0, 0)),
        ],
        out_specs=[
            pl.BlockSpec((ROW_BLK, D_H), lambda i: (i, 0)),
            pl.BlockSpec((ROW_BLK, D_H), lambda i: (i, 0)),
            pl.BlockSpec((ROW_BLK, 1), lambda i: (i, 0)),
        ],
        out_shape=[
            jax.ShapeDtypeStruct((N_NODES, D_H), jnp.float32),
            jax.ShapeDtypeStruct((N_NODES, D_H), jnp.float32),
            jax.ShapeDtypeStruct((N_NODES, 1), jnp.float32),
        ],
    )(agg, degp, b, r, w_l, w_r)


def _combine2_body(agg_ref, rdeg_ref, b_ref, r_ref, o_ref):
    o_ref[...] = jnp.maximum(
        (agg_ref[0] + agg_ref[1]) * rdeg_ref[...] + b_ref[...] + r_ref[...],
        0.0)


def _combine2(agg, rdeg, b, r):
    return pl.pallas_call(
        _combine2_body,
        grid=(N_NODES // ROW_BLK,),
        in_specs=[
            pl.BlockSpec((NC, ROW_BLK, D_H), lambda i: (0, i, 0)),
            pl.BlockSpec((ROW_BLK, 1), lambda i: (i, 0)),
            pl.BlockSpec((1, D_H), lambda i: (0, 0)),
            pl.BlockSpec((ROW_BLK, D_H), lambda i: (i, 0)),
        ],
        out_specs=pl.BlockSpec((ROW_BLK, D_H), lambda i: (i, 0)),
        out_shape=jax.ShapeDtypeStruct((N_NODES, D_H), jnp.float32),
    )(agg, rdeg, b, r)


def _make_sc_aggregate(with_deg):
    mesh = plsc.VectorSubcoreMesh(core_axis_name="c", subcore_axis_name="s")
    out_type = [jax.ShapeDtypeStruct((NC, N_NODES, D_H), jnp.float32)]
    scratch = [
        pltpu.VMEM((K,), jnp.int32),
        pltpu.VMEM((K,), jnp.int32),
        pltpu.VMEM((K, D_H), jnp.float32),
        pltpu.VMEM_SHARED((N_NODES, D_H), jnp.float32),
        pltpu.SemaphoreType.DMA,
    ]
    if with_deg:
        out_type.append(
            jax.ShapeDtypeStruct((N_NODES // ROW_BLK, NW, ROW_BLK),
                                 jnp.float32))
        scratch.append(pltpu.VMEM((N_NODES,), jnp.float32))

    def body(p_hbm, src_hbm, dst_hbm, zeros_hbm, *rest):
        if with_deg:
            agg_out, deg_out, sidx, didx, rows, aggsh, sem, ldeg = rest
        else:
            (agg_out,), (sidx, didx, rows, aggsh, sem) = rest[:1], rest[1:]
        c = lax.axis_index("c")
        s = lax.axis_index("s")
        wid = c * NS + s

        pltpu.sync_copy(zeros_hbm.at[pl.ds(s * RPS, RPS)],
                        aggsh.at[pl.ds(s * RPS, RPS)])
        if with_deg:
            z16 = jnp.zeros((16,), jnp.float32)

            def zero_deg(i, carry):
                ldeg[pl.ds(i * 16, 16)] = z16
                return carry
            lax.fori_loop(0, N_NODES // 16, zero_deg, 0)
        plsc.subcore_barrier()

        ones16 = jnp.ones((16,), jnp.float32)

        def step(j, carry):
            chunk = wid * CHUNKS + j
            pltpu.sync_copy(src_hbm.at[chunk], sidx)
            pltpu.sync_copy(dst_hbm.at[chunk], didx)
            pltpu.async_copy(p_hbm.at[sidx], rows, sem).wait()
            pltpu.sync_copy(rows, aggsh.at[didx], add=True)
            if with_deg:
                for t in range(K // 16):
                    plsc.addupdate_scatter(
                        ldeg, [didx[pl.ds(t * 16, 16)]], ones16)
            return carry
        lax.fori_loop(0, CHUNKS, step, 0)

        plsc.subcore_barrier()

        pltpu.sync_copy(aggsh.at[pl.ds(s * RPS, RPS)],
                        agg_out.at[c, pl.ds(s * RPS, RPS)])
        if with_deg:
            for b in range(N_NODES // ROW_BLK):
                pltpu.sync_copy(ldeg.at[pl.ds(b * ROW_BLK, ROW_BLK)],
                                deg_out.at[b, wid])

    return pl.kernel(
        body,
        out_type=tuple(out_type) if with_deg else out_type[0],
        mesh=mesh,
        scratch_types=scratch,
        compiler_params=pltpu.CompilerParams(use_tc_tiling_on_sc=False,
                                             needs_layout_passes=False),
    )


_sc_aggregate_deg = _make_sc_aggregate(with_deg=True)
_sc_aggregate = _make_sc_aggregate(with_deg=False)


def kernel(x, edge_index, W1_l, b1, W1_r, W2_l, b2, W2_r):
    ei = edge_index.astype(jnp.int32)
    src = ei[0].reshape(NW * CHUNKS, K)
    dst = ei[1].reshape(NW * CHUNKS, K)
    zeros = jnp.zeros((N_NODES, D_H), jnp.float32)
    b1r = b1.reshape(1, D_H)
    b2r = b2.reshape(1, D_H)

    p1, r1 = _project(x, W1_l, W1_r)
    agg1, degp = _sc_aggregate_deg(p1, src, dst, zeros)
    p2, r2, rdeg = _combine1(agg1, degp, b1r, r1, W2_l, W2_r)
    agg2 = _sc_aggregate(p2, src, dst, zeros)
    return _combine2(agg2, rdeg, b2r, r2)


# speedup vs baseline: 12.3769x; 1.7865x over previous
import functools

import jax
import jax.numpy as jnp
from jax import lax
from jax.experimental import pallas as pl
from jax.experimental.pallas import tpu as pltpu
from jax.experimental.pallas import tpu_sc as plsc

N_NODES = 10000
N_EDGES = 320000
D_IN = 128
D_H = 32

NC, NS = 2, 16
NW = NC * NS
K = 80
CHUNKS = N_EDGES // (NW * K)
RPS = N_NODES // NS

ROW_BLK = 2000


def _project_body(x_ref, wl_ref, wr_ref, p_ref, r_ref):
    xb = x_ref[...]
    p_ref[...] = jnp.dot(xb, wl_ref[...], preferred_element_type=jnp.float32)
    r_ref[...] = jnp.dot(xb, wr_ref[...], preferred_element_type=jnp.float32)


def _project(x, w_l, w_r):
    n, d = x.shape
    return pl.pallas_call(
        _project_body,
        grid=(n // ROW_BLK,),
        in_specs=[
            pl.BlockSpec((ROW_BLK, d), lambda i: (i, 0)),
            pl.BlockSpec((d, D_H), lambda i: (0, 0)),
            pl.BlockSpec((d, D_H), lambda i: (0, 0)),
        ],
        out_specs=[
            pl.BlockSpec((ROW_BLK, D_H), lambda i: (i, 0)),
            pl.BlockSpec((ROW_BLK, D_H), lambda i: (i, 0)),
        ],
        out_shape=[jax.ShapeDtypeStruct((n, D_H), jnp.float32)] * 2,
    )(x, w_l, w_r)


def _combine1_body(agg_ref, degp_ref, b_ref, r_ref, wl_ref, wr_ref,
                   p2_ref, r2_ref, rdeg_ref):
    ones = jnp.ones((NW, 1), jnp.float32)
    deg = lax.dot_general(degp_ref[0], ones, (((0,), (0,)), ((), ())),
                          preferred_element_type=jnp.float32)
    rdeg = 1.0 / jnp.maximum(deg, 1.0)
    h = jnp.maximum(
        (agg_ref[0] + agg_ref[1]) * rdeg + b_ref[...] + r_ref[...], 0.0)
    p2_ref[...] = jnp.dot(h, wl_ref[...], preferred_element_type=jnp.float32)
    r2_ref[...] = jnp.dot(h, wr_ref[...], preferred_element_type=jnp.float32)
    rdeg_ref[...] = rdeg


def _combine1(agg, degp, b, r, w_l, w_r):
    return pl.pallas_call(
        _combine1_body,
        grid=(N_NODES // ROW_BLK,),
        in_specs=[
            pl.BlockSpec((NC, ROW_BLK, D_H), lambda i: (0, i, 0)),
            pl.BlockSpec((1, NW, ROW_BLK), lambda i: (i, 0, 0)),
            pl.BlockSpec((1, D_H), lambda i: (0, 0)),
            pl.BlockSpec((ROW_BLK, D_H), lambda i: (i, 0)),
            pl.BlockSpec((D_H, D_H), lambda i: (0, 0)),
            pl.BlockSpec((D_H, D_H), lambda i: (0, 0)),
        ],
        out_specs=[
            pl.BlockSpec((ROW_BLK, D_H), lambda i: (i, 0)),
            pl.BlockSpec((ROW_BLK, D_H), lambda i: (i, 0)),
            pl.BlockSpec((ROW_BLK, 1), lambda i: (i, 0)),
        ],
        out_shape=[
            jax.ShapeDtypeStruct((N_NODES, D_H), jnp.float32),
            jax.ShapeDtypeStruct((N_NODES, D_H), jnp.float32),
            jax.ShapeDtypeStruct((N_NODES, 1), jnp.float32),
        ],
    )(agg, degp, b, r, w_l, w_r)


def _combine2_body(agg_ref, rdeg_ref, b_ref, r_ref, o_ref):
    o_ref[...] = jnp.maximum(
        (agg_ref[0] + agg_ref[1]) * rdeg_ref[...] + b_ref[...] + r_ref[...],
        0.0)


def _combine2(agg, rdeg, b, r):
    return pl.pallas_call(
        _combine2_body,
        grid=(N_NODES // ROW_BLK,),
        in_specs=[
            pl.BlockSpec((NC, ROW_BLK, D_H), lambda i: (0, i, 0)),
            pl.BlockSpec((ROW_BLK, 1), lambda i: (i, 0)),
            pl.BlockSpec((1, D_H), lambda i: (0, 0)),
            pl.BlockSpec((ROW_BLK, D_H), lambda i: (i, 0)),
        ],
        out_specs=pl.BlockSpec((ROW_BLK, D_H), lambda i: (i, 0)),
        out_shape=jax.ShapeDtypeStruct((N_NODES, D_H), jnp.float32),
    )(agg, rdeg, b, r)


def _make_sc_aggregate(with_deg):
    mesh = plsc.VectorSubcoreMesh(core_axis_name="c", subcore_axis_name="s")
    out_type = [jax.ShapeDtypeStruct((NC, N_NODES, D_H), jnp.float32)]
    scratch = [
        pltpu.VMEM((2, K), jnp.int32),
        pltpu.VMEM((2, K), jnp.int32),
        pltpu.VMEM((K, D_H), jnp.float32),
        pltpu.VMEM((K, D_H), jnp.float32),
        pltpu.VMEM_SHARED((N_NODES, D_H), jnp.float32),
        pltpu.SemaphoreType.DMA,
        pltpu.SemaphoreType.DMA,
        pltpu.SemaphoreType.DMA,
        pltpu.SemaphoreType.DMA,
    ]
    if with_deg:
        out_type.append(
            jax.ShapeDtypeStruct((N_NODES // ROW_BLK, NW, ROW_BLK),
                                 jnp.float32))
        scratch.append(pltpu.VMEM((N_NODES,), jnp.float32))

    def body(p_hbm, idx_hbm, zeros_hbm, *rest):
        if with_deg:
            (agg_out, deg_out, i0, i1, r0, r1, aggsh,
             si0, si1, sg0, sg1, ldeg) = rest
        else:
            (agg_out, i0, i1, r0, r1, aggsh, si0, si1, sg0, sg1) = rest
        idxb = (i0, i1)
        rows = (r0, r1)
        sem_i = (si0, si1)
        sem_g = (sg0, sg1)
        c = lax.axis_index("c")
        s = lax.axis_index("s")
        wid = c * NS + s

        pltpu.sync_copy(zeros_hbm.at[pl.ds(s * RPS, RPS)],
                        aggsh.at[pl.ds(s * RPS, RPS)])
        if with_deg:
            z16 = jnp.zeros((16,), jnp.float32)

            def zero_deg(i, carry):
                ldeg[pl.ds(i * 16, 16)] = z16
                return carry
            lax.fori_loop(0, N_NODES // 16, zero_deg, 0)
        plsc.subcore_barrier()

        ones16 = jnp.ones((16,), jnp.float32)
        base = wid * CHUNKS

        pltpu.sync_copy(idx_hbm.at[base], idxb[0])
        pltpu.async_copy(idx_hbm.at[base + 1], idxb[1], sem_i[1])
        pltpu.async_copy(p_hbm.at[idxb[0].at[0]], rows[0], sem_g[0])

        def chunk_step(j, p):
            @pl.when(j + 1 < CHUNKS)
            def _prefetch():
                q = 1 - p
                pltpu.make_async_copy(
                    idx_hbm.at[base + j + 1], idxb[q], sem_i[q]).wait()
                pltpu.async_copy(p_hbm.at[idxb[q].at[0]], rows[q], sem_g[q])

            pltpu.make_async_copy(
                p_hbm.at[idxb[p].at[0]], rows[p], sem_g[p]).wait()
            pltpu.sync_copy(rows[p], aggsh.at[idxb[p].at[1]], add=True)
            if with_deg:
                for t in range(K // 16):
                    plsc.addupdate_scatter(
                        ldeg, [idxb[p][1, pl.ds(t * 16, 16)]], ones16)

            @pl.when(j + 2 < CHUNKS)
            def _next_idx():
                pltpu.async_copy(idx_hbm.at[base + j + 2], idxb[p], sem_i[p])

        def step(i, carry):
            j = i * 2

            @pl.when(j < CHUNKS)
            def _even():
                chunk_step(j, 0)

            @pl.when(j + 1 < CHUNKS)
            def _odd():
                chunk_step(j + 1, 1)
            return carry
        lax.fori_loop(0, (CHUNKS + 1) // 2, step, 0)

        plsc.subcore_barrier()

        pltpu.sync_copy(aggsh.at[pl.ds(s * RPS, RPS)],
                        agg_out.at[c, pl.ds(s * RPS, RPS)])
        if with_deg:
            for b in range(N_NODES // ROW_BLK):
                pltpu.sync_copy(ldeg.at[pl.ds(b * ROW_BLK, ROW_BLK)],
                                deg_out.at[b, wid])

    return pl.kernel(
        body,
        out_type=tuple(out_type) if with_deg else out_type[0],
        mesh=mesh,
        scratch_types=scratch,
        compiler_params=pltpu.CompilerParams(use_tc_tiling_on_sc=False,
                                             needs_layout_passes=False),
    )


_sc_aggregate_deg = _make_sc_aggregate(with_deg=True)
_sc_aggregate = _make_sc_aggregate(with_deg=False)


def kernel(x, edge_index, W1_l, b1, W1_r, W2_l, b2, W2_r):
    ei = edge_index.astype(jnp.int32)
    idx2 = jnp.stack(
        [ei[0].reshape(NW * CHUNKS, K), ei[1].reshape(NW * CHUNKS, K)],
        axis=1)
    zeros = jnp.zeros((N_NODES, D_H), jnp.float32)
    b1r = b1.reshape(1, D_H)
    b2r = b2.reshape(1, D_H)

    p1, r1 = _project(x, W1_l, W1_r)
    agg1, degp = _sc_aggregate_deg(p1, idx2, zeros)
    p2, r2, rdeg = _combine1(agg1, degp, b1r, r1, W2_l, W2_r)
    agg2 = _sc_aggregate(p2, idx2, zeros)
    return _combine2(agg2, rdeg, b2r, r2)


# speedup vs baseline: 15.4286x; 1.2466x over previous
import functools

import jax
import jax.numpy as jnp
from jax import lax
from jax.experimental import pallas as pl
from jax.experimental.pallas import tpu as pltpu
from jax.experimental.pallas import tpu_sc as plsc

N_NODES = 10000
N_EDGES = 320000
D_IN = 128
D_H = 32

NC, NS = 2, 16
NW = NC * NS
K = 80
CHUNKS = N_EDGES // (NW * K)
RPS = N_NODES // NS

ROW_BLK = 2000


def _project_body(x_ref, wl_ref, wr_ref, p_ref, r_ref):
    xb = x_ref[...]
    p_ref[...] = jnp.dot(xb, wl_ref[...], preferred_element_type=jnp.float32)
    r_ref[...] = jnp.dot(xb, wr_ref[...], preferred_element_type=jnp.float32)


def _project(x, w_l, w_r):
    n, d = x.shape
    return pl.pallas_call(
        _project_body,
        grid=(n // ROW_BLK,),
        in_specs=[
            pl.BlockSpec((ROW_BLK, d), lambda i: (i, 0)),
            pl.BlockSpec((d, D_H), lambda i: (0, 0)),
            pl.BlockSpec((d, D_H), lambda i: (0, 0)),
        ],
        out_specs=[
            pl.BlockSpec((ROW_BLK, D_H), lambda i: (i, 0)),
            pl.BlockSpec((ROW_BLK, D_H), lambda i: (i, 0)),
        ],
        out_shape=[jax.ShapeDtypeStruct((n, D_H), jnp.float32)] * 2,
    )(x, w_l, w_r)


def _combine1_body(agg_ref, degp_ref, b_ref, r_ref, wl_ref, wr_ref,
                   p2_ref, r2_ref, rdeg_ref):
    ones = jnp.ones((NW, 1), jnp.float32)
    deg = lax.dot_general(degp_ref[0], ones, (((0,), (0,)), ((), ())),
                          preferred_element_type=jnp.float32)
    rdeg = 1.0 / jnp.maximum(deg, 1.0)
    h = jnp.maximum(
        (agg_ref[0] + agg_ref[1]) * rdeg + b_ref[...] + r_ref[...], 0.0)
    p2_ref[...] = jnp.dot(h, wl_ref[...], preferred_element_type=jnp.float32)
    r2_ref[...] = jnp.dot(h, wr_ref[...], preferred_element_type=jnp.float32)
    rdeg_ref[...] = rdeg


def _combine1(agg, degp, b, r, w_l, w_r):
    return pl.pallas_call(
        _combine1_body,
        grid=(N_NODES // ROW_BLK,),
        in_specs=[
            pl.BlockSpec((NC, ROW_BLK, D_H), lambda i: (0, i, 0)),
            pl.BlockSpec((1, NW, ROW_BLK), lambda i: (i, 0, 0)),
            pl.BlockSpec((1, D_H), lambda i: (0, 0)),
            pl.BlockSpec((ROW_BLK, D_H), lambda i: (i, 0)),
            pl.BlockSpec((D_H, D_H), lambda i: (0, 0)),
            pl.BlockSpec((D_H, D_H), lambda i: (0, 0)),
        ],
        out_specs=[
            pl.BlockSpec((ROW_BLK, D_H), lambda i: (i, 0)),
            pl.BlockSpec((ROW_BLK, D_H), lambda i: (i, 0)),
            pl.BlockSpec((ROW_BLK, 1), lambda i: (i, 0)),
        ],
        out_shape=[
            jax.ShapeDtypeStruct((N_NODES, D_H), jnp.float32),
            jax.ShapeDtypeStruct((N_NODES, D_H), jnp.float32),
            jax.ShapeDtypeStruct((N_NODES, 1), jnp.float32),
        ],
    )(agg, degp, b, r, w_l, w_r)


def _combine2_body(agg_ref, rdeg_ref, b_ref, r_ref, o_ref):
    o_ref[...] = jnp.maximum(
        (agg_ref[0] + agg_ref[1]) * rdeg_ref[...] + b_ref[...] + r_ref[...],
        0.0)


def _combine2(agg, rdeg, b, r):
    return pl.pallas_call(
        _combine2_body,
        grid=(N_NODES // ROW_BLK,),
        in_specs=[
            pl.BlockSpec((NC, ROW_BLK, D_H), lambda i: (0, i, 0)),
            pl.BlockSpec((ROW_BLK, 1), lambda i: (i, 0)),
            pl.BlockSpec((1, D_H), lambda i: (0, 0)),
            pl.BlockSpec((ROW_BLK, D_H), lambda i: (i, 0)),
        ],
        out_specs=pl.BlockSpec((ROW_BLK, D_H), lambda i: (i, 0)),
        out_shape=jax.ShapeDtypeStruct((N_NODES, D_H), jnp.float32),
    )(agg, rdeg, b, r)


def _make_sc_aggregate(with_deg):
    mesh = plsc.VectorSubcoreMesh(core_axis_name="c", subcore_axis_name="s")
    out_type = [jax.ShapeDtypeStruct((NC, N_NODES, D_H), jnp.float32)]
    scratch = (
        [pltpu.VMEM((2, K), jnp.int32) for _ in range(4)]
        + [pltpu.VMEM((K, D_H), jnp.float32) for _ in range(2)]
        + [pltpu.VMEM_SHARED((N_NODES, D_H), jnp.float32)]
        + [pltpu.SemaphoreType.DMA] * 8
    )
    if with_deg:
        out_type.append(
            jax.ShapeDtypeStruct((N_NODES // ROW_BLK, NW, ROW_BLK),
                                 jnp.float32))
        scratch.append(pltpu.VMEM((N_NODES,), jnp.float32))

    def body(p_hbm, idx_hbm, zeros_hbm, *rest):
        if with_deg:
            agg_out, deg_out = rest[0], rest[1]
            rest = rest[2:]
        else:
            agg_out = rest[0]
            rest = rest[1:]
        idxb = rest[0:4]
        rows = rest[4:6]
        aggsh = rest[6]
        sem_i = rest[7:11]
        sem_g = rest[11:13]
        sem_s = rest[13:15]
        if with_deg:
            ldeg = rest[15]
        c = lax.axis_index("c")
        s = lax.axis_index("s")
        wid = c * NS + s

        pltpu.sync_copy(zeros_hbm.at[pl.ds(s * RPS, RPS)],
                        aggsh.at[pl.ds(s * RPS, RPS)])
        if with_deg:
            z16 = jnp.zeros((16,), jnp.float32)

            def zero_deg(i, carry):
                ldeg[pl.ds(i * 16, 16)] = z16
                return carry
            lax.fori_loop(0, N_NODES // 16, zero_deg, 0)
        plsc.subcore_barrier()

        ones16 = jnp.ones((16,), jnp.float32)
        base = wid * CHUNKS

        pltpu.sync_copy(idx_hbm.at[base], idxb[0])
        pltpu.async_copy(idx_hbm.at[base + 1], idxb[1], sem_i[1])
        pltpu.async_copy(idx_hbm.at[base + 2], idxb[2], sem_i[2])
        pltpu.async_copy(p_hbm.at[idxb[0].at[0]], rows[0], sem_g[0])

        def chunk_step(j, b):
            r = b
            p = b % 2
            q = 1 - p
            rp = (b - 1) % 4
            rn = (b + 3) % 4

            @pl.when(j + 1 < CHUNKS)
            def _launch_next_gather():
                pltpu.make_async_copy(
                    idx_hbm.at[base + j + 1], idxb[(b + 1) % 4],
                    sem_i[(b + 1) % 4]).wait()

                @pl.when(j >= 1)
                def _drain_prev_scatter():
                    pltpu.make_async_copy(
                        rows[q], aggsh.at[idxb[rp].at[1]], sem_s[q]).wait()
                pltpu.async_copy(
                    p_hbm.at[idxb[(b + 1) % 4].at[0]], rows[q], sem_g[q])

            pltpu.make_async_copy(
                p_hbm.at[idxb[r].at[0]], rows[p], sem_g[p]).wait()
            pltpu.async_copy(rows[p], aggsh.at[idxb[r].at[1]], sem_s[p],
                             add=True)
            if with_deg:
                for t in range(K // 16):
                    plsc.addupdate_scatter(
                        ldeg, [idxb[r][1, pl.ds(t * 16, 16)]], ones16)

            @pl.when(j + 3 < CHUNKS)
            def _next_idx():
                pltpu.async_copy(idx_hbm.at[base + j + 3], idxb[rn],
                                 sem_i[rn])

        def step(i, carry):
            j0 = i * 4
            for b in range(4):
                @pl.when(j0 + b < CHUNKS)
                def _do(jb=j0 + b, bb=b):
                    chunk_step(jb, bb)
            return carry
        lax.fori_loop(0, (CHUNKS + 3) // 4, step, 0)

        last = CHUNKS - 1
        pltpu.make_async_copy(
            rows[(last - 1) % 2], aggsh.at[idxb[(last - 1) % 4].at[1]],
            sem_s[(last - 1) % 2]).wait()
        pltpu.make_async_copy(
            rows[last % 2], aggsh.at[idxb[last % 4].at[1]],
            sem_s[last % 2]).wait()

        plsc.subcore_barrier()

        pltpu.sync_copy(aggsh.at[pl.ds(s * RPS, RPS)],
                        agg_out.at[c, pl.ds(s * RPS, RPS)])
        if with_deg:
            for b in range(N_NODES // ROW_BLK):
                pltpu.sync_copy(ldeg.at[pl.ds(b * ROW_BLK, ROW_BLK)],
                                deg_out.at[b, wid])

    return pl.kernel(
        body,
        out_type=tuple(out_type) if with_deg else out_type[0],
        mesh=mesh,
        scratch_types=scratch,
        compiler_params=pltpu.CompilerParams(use_tc_tiling_on_sc=False,
                                             needs_layout_passes=False),
    )


_sc_aggregate_deg = _make_sc_aggregate(with_deg=True)
_sc_aggregate = _make_sc_aggregate(with_deg=False)


def kernel(x, edge_index, W1_l, b1, W1_r, W2_l, b2, W2_r):
    ei = edge_index.astype(jnp.int32)
    idx2 = jnp.stack(
        [ei[0].reshape(NW * CHUNKS, K), ei[1].reshape(NW * CHUNKS, K)],
        axis=1)
    zeros = jnp.zeros((N_NODES, D_H), jnp.float32)
    b1r = b1.reshape(1, D_H)
    b2r = b2.reshape(1, D_H)

    p1, r1 = _project(x, W1_l, W1_r)
    agg1, degp = _sc_aggregate_deg(p1, idx2, zeros)
    p2, r2, rdeg = _combine1(agg1, degp, b1r, r1, W2_l, W2_r)
    agg2 = _sc_aggregate(p2, idx2, zeros)
    return _combine2(agg2, rdeg, b2r, r2)


# speedup vs baseline: 16.3251x; 1.0581x over previous
import functools

import jax
import jax.numpy as jnp
from jax import lax
from jax.experimental import pallas as pl
from jax.experimental.pallas import tpu as pltpu
from jax.experimental.pallas import tpu_sc as plsc

N_NODES = 10000
N_EDGES = 320000
D_IN = 128
D_H = 32

NC, NS = 2, 16
NW = NC * NS
K = 80
CHUNKS = N_EDGES // (NW * K)
N_PAD = 10240
RPS = N_PAD // NS

ROW_BLK = 2000


def _project_body(x_ref, wl_ref, wr_ref, p_ref, r_ref):
    xb = x_ref[...]
    p_ref[...] = jnp.dot(xb, wl_ref[...], preferred_element_type=jnp.float32)
    r_ref[...] = jnp.dot(xb, wr_ref[...], preferred_element_type=jnp.float32)


def _project(x, w_l, w_r):
    n, d = x.shape
    return pl.pallas_call(
        _project_body,
        grid=(n // ROW_BLK,),
        in_specs=[
            pl.BlockSpec((ROW_BLK, d), lambda i: (i, 0)),
            pl.BlockSpec((d, D_H), lambda i: (0, 0)),
            pl.BlockSpec((d, D_H), lambda i: (0, 0)),
        ],
        out_specs=[
            pl.BlockSpec((ROW_BLK, D_H), lambda i: (i, 0)),
            pl.BlockSpec((ROW_BLK, D_H), lambda i: (i, 0)),
        ],
        out_shape=[jax.ShapeDtypeStruct((n, D_H), jnp.float32)] * 2,
    )(x, w_l, w_r)


def _combine1_body(agg_ref, degp_ref, b_ref, r_ref, wl_ref, wr_ref,
                   p2_ref, r2_ref, rdeg_ref):
    deg = degp_ref[0][:, :1] + degp_ref[1][:, :1]
    rdeg = 1.0 / jnp.maximum(deg, 1.0)
    h = jnp.maximum(
        (agg_ref[0] + agg_ref[1]) * rdeg + b_ref[...] + r_ref[...], 0.0)
    p2_ref[...] = jnp.dot(h, wl_ref[...], preferred_element_type=jnp.float32)
    r2_ref[...] = jnp.dot(h, wr_ref[...], preferred_element_type=jnp.float32)
    rdeg_ref[...] = rdeg


def _combine1(agg, degp, b, r, w_l, w_r):
    return pl.pallas_call(
        _combine1_body,
        grid=(N_NODES // ROW_BLK,),
        in_specs=[
            pl.BlockSpec((NC, ROW_BLK, D_H), lambda i: (0, i, 0)),
            pl.BlockSpec((NC, ROW_BLK, 16), lambda i: (0, i, 0)),
            pl.BlockSpec((D_H,), lambda i: (0,)),
            pl.BlockSpec((ROW_BLK, D_H), lambda i: (i, 0)),
            pl.BlockSpec((D_H, D_H), lambda i: (0, 0)),
            pl.BlockSpec((D_H, D_H), lambda i: (0, 0)),
        ],
        out_specs=[
            pl.BlockSpec((ROW_BLK, D_H), lambda i: (i, 0)),
            pl.BlockSpec((ROW_BLK, D_H), lambda i: (i, 0)),
            pl.BlockSpec((ROW_BLK, 1), lambda i: (i, 0)),
        ],
        out_shape=[
            jax.ShapeDtypeStruct((N_NODES, D_H), jnp.float32),
            jax.ShapeDtypeStruct((N_NODES, D_H), jnp.float32),
            jax.ShapeDtypeStruct((N_NODES, 1), jnp.float32),
        ],
    )(agg, degp, b, r, w_l, w_r)


def _combine2_body(agg_ref, rdeg_ref, b_ref, r_ref, o_ref):
    o_ref[...] = jnp.maximum(
        (agg_ref[0] + agg_ref[1]) * rdeg_ref[...] + b_ref[...] + r_ref[...],
        0.0)


def _combine2(agg, rdeg, b, r):
    return pl.pallas_call(
        _combine2_body,
        grid=(N_NODES // ROW_BLK,),
        in_specs=[
            pl.BlockSpec((NC, ROW_BLK, D_H), lambda i: (0, i, 0)),
            pl.BlockSpec((ROW_BLK, 1), lambda i: (i, 0)),
            pl.BlockSpec((D_H,), lambda i: (0,)),
            pl.BlockSpec((ROW_BLK, D_H), lambda i: (i, 0)),
        ],
        out_specs=pl.BlockSpec((ROW_BLK, D_H), lambda i: (i, 0)),
        out_shape=jax.ShapeDtypeStruct((N_NODES, D_H), jnp.float32),
    )(agg, rdeg, b, r)


def _make_sc_aggregate(with_deg):
    mesh = plsc.VectorSubcoreMesh(core_axis_name="c", subcore_axis_name="s")
    out_type = [jax.ShapeDtypeStruct((NC, N_PAD, D_H), jnp.float32)]
    scratch = (
        [pltpu.VMEM((K,), jnp.int32) for _ in range(4)]
        + [pltpu.VMEM((K,), jnp.int32) for _ in range(4)]
        + [pltpu.VMEM((K, D_H), jnp.float32) for _ in range(2)]
        + [pltpu.VMEM_SHARED((N_PAD, D_H), jnp.float32)]
        + [pltpu.SemaphoreType.DMA] * 8
    )
    if with_deg:
        out_type.append(jax.ShapeDtypeStruct((NC, N_PAD, 16), jnp.float32))
        scratch += [
            pltpu.VMEM((N_PAD,), jnp.float32),
            pltpu.VMEM_SHARED((NS, N_PAD), jnp.float32),
            pltpu.VMEM((RPS + 16,), jnp.float32),
            pltpu.VMEM((RPS,), jnp.float32),
            pltpu.VMEM((RPS, 16), jnp.float32),
        ]

    def body(p_hbm, eflat_hbm, zeros_hbm, *rest):
        if with_deg:
            agg_out, deg_out = rest[0], rest[1]
            rest = rest[2:]
        else:
            agg_out = rest[0]
            rest = rest[1:]
        sidx = rest[0:4]
        didx = rest[4:8]
        rows = rest[8:10]
        aggsh = rest[10]
        sem_i = rest[11:15]
        sem_g = rest[15:17]
        sem_s = rest[17:19]
        if with_deg:
            ldeg, parts_sh, racc, rtmp, rspread = rest[19:24]
        c = lax.axis_index("c")
        s = lax.axis_index("s")
        wid = c * NS + s

        pltpu.sync_copy(zeros_hbm.at[pl.ds(s * RPS, RPS)],
                        aggsh.at[pl.ds(s * RPS, RPS)])
        if with_deg:
            z16 = jnp.zeros((16,), jnp.float32)

            def zero_deg(i, carry):
                ldeg[pl.ds(i * 16, 16)] = z16
                return carry
            lax.fori_loop(0, N_PAD // 16, zero_deg, 0)
        plsc.subcore_barrier()

        ones16 = jnp.ones((16,), jnp.float32)
        ebase = wid * CHUNKS * K

        def fire_idx(j, slot):
            pltpu.async_copy(eflat_hbm.at[pl.ds(ebase + j * K, K)],
                             sidx[slot], sem_i[slot])
            pltpu.async_copy(
                eflat_hbm.at[pl.ds(N_EDGES + ebase + j * K, K)],
                didx[slot], sem_i[slot])

        def wait_idx(j, slot):
            pltpu.make_async_copy(eflat_hbm.at[pl.ds(ebase + j * K, K)],
                                  sidx[slot], sem_i[slot]).wait()
            pltpu.make_async_copy(
                eflat_hbm.at[pl.ds(N_EDGES + ebase + j * K, K)],
                didx[slot], sem_i[slot]).wait()

        fire_idx(0, 0)
        wait_idx(0, 0)
        fire_idx(1, 1)
        fire_idx(2, 2)
        pltpu.async_copy(p_hbm.at[sidx[0]], rows[0], sem_g[0])

        def chunk_step(j, b):
            r = b
            p = b % 2
            q = 1 - p
            rp = (b - 1) % 4
            rn = (b + 3) % 4

            @pl.when(j + 1 < CHUNKS)
            def _launch_next_gather():
                wait_idx(j + 1, (b + 1) % 4)

                @pl.when(j >= 1)
                def _drain_prev_scatter():
                    pltpu.make_async_copy(
                        rows[q], aggsh.at[didx[rp]], sem_s[q]).wait()
                pltpu.async_copy(
                    p_hbm.at[sidx[(b + 1) % 4]], rows[q], sem_g[q])

            pltpu.make_async_copy(
                p_hbm.at[sidx[r]], rows[p], sem_g[p]).wait()
            pltpu.async_copy(rows[p], aggsh.at[didx[r]], sem_s[p], add=True)
            if with_deg:
                for t in range(K // 16):
                    plsc.addupdate_scatter(
                        ldeg, [didx[r][pl.ds(t * 16, 16)]], ones16)

            @pl.when(j + 3 < CHUNKS)
            def _next_idx():
                fire_idx(j + 3, rn)

        def step(i, carry):
            j0 = i * 4
            for b in range(4):
                @pl.when(j0 + b < CHUNKS)
                def _do(jb=j0 + b, bb=b):
                    chunk_step(jb, bb)
            return carry
        lax.fori_loop(0, (CHUNKS + 3) // 4, step, 0)

        last = CHUNKS - 1
        pltpu.make_async_copy(
            rows[(last - 1) % 2], aggsh.at[didx[(last - 1) % 4]],
            sem_s[(last - 1) % 2]).wait()
        pltpu.make_async_copy(
            rows[last % 2], aggsh.at[didx[last % 4]],
            sem_s[last % 2]).wait()

        plsc.subcore_barrier()

        pltpu.sync_copy(aggsh.at[pl.ds(s * RPS, RPS)],
                        agg_out.at[c, pl.ds(s * RPS, RPS)])

        if with_deg:
            pltpu.sync_copy(ldeg, parts_sh.at[s])
            plsc.subcore_barrier()
            seg = s * RPS
            pltpu.sync_copy(parts_sh.at[0, pl.ds(seg, RPS)],
                            racc.at[pl.ds(0, RPS)])

            def red_k(k, carry):
                pltpu.sync_copy(parts_sh.at[k, pl.ds(seg, RPS)], rtmp)

                def red_v(t, carry2):
                    sl = pl.ds(t * 16, 16)
                    racc[sl] = racc[sl] + rtmp[sl]
                    return carry2
                lax.fori_loop(0, RPS // 16, red_v, 0)
                return carry
            lax.fori_loop(1, NS, red_k, 0)

            def spread(i, carry):
                rspread[i, pl.ds(0, 16)] = racc[pl.ds(i, 16)]
                return carry
            lax.fori_loop(0, RPS, spread, 0)
            pltpu.sync_copy(rspread, deg_out.at[c, pl.ds(seg, RPS)])

    return pl.kernel(
        body,
        out_type=tuple(out_type) if with_deg else out_type[0],
        mesh=mesh,
        scratch_types=scratch,
        compiler_params=pltpu.CompilerParams(use_tc_tiling_on_sc=False,
                                             needs_layout_passes=False),
    )


_sc_aggregate_deg = _make_sc_aggregate(with_deg=True)
_sc_aggregate = _make_sc_aggregate(with_deg=False)


def kernel(x, edge_index, W1_l, b1, W1_r, W2_l, b2, W2_r):
    eflat = edge_index.astype(jnp.int32).reshape(2 * N_EDGES)
    zeros = jnp.zeros((N_PAD, D_H), jnp.float32)

    p1, r1 = _project(x, W1_l, W1_r)
    agg1, degp = _sc_aggregate_deg(p1, eflat, zeros)
    p2, r2, rdeg = _combine1(agg1, degp, b1, r1, W2_l, W2_r)
    agg2 = _sc_aggregate(p2, eflat, zeros)
    return _combine2(agg2, rdeg, b2, r2)


# speedup vs baseline: 18.7280x; 1.1472x over previous
import functools

import jax
import jax.numpy as jnp
from jax import lax
from jax.experimental import pallas as pl
from jax.experimental.pallas import tpu as pltpu
from jax.experimental.pallas import tpu_sc as plsc

N_NODES = 10000
N_EDGES = 320000
D_IN = 128
D_H = 32

NC, NS = 2, 16
NW = NC * NS
K = 128
CHUNKS = N_EDGES // (NW * K)
EXTRA = (N_EDGES // K) % NW
N_PAD = 10240
RPS = N_PAD // NS

ROW_BLK = 2000


def _project_body(x_ref, wl_ref, wr_ref, p_ref, r_ref):
    xb = x_ref[...]
    p_ref[...] = jnp.dot(xb, wl_ref[...], preferred_element_type=jnp.float32)
    r_ref[...] = jnp.dot(xb, wr_ref[...], preferred_element_type=jnp.float32)


def _project(x, w_l, w_r):
    n, d = x.shape
    return pl.pallas_call(
        _project_body,
        grid=(n // ROW_BLK,),
        in_specs=[
            pl.BlockSpec((ROW_BLK, d), lambda i: (i, 0)),
            pl.BlockSpec((d, D_H), lambda i: (0, 0)),
            pl.BlockSpec((d, D_H), lambda i: (0, 0)),
        ],
        out_specs=[
            pl.BlockSpec((ROW_BLK, D_H), lambda i: (i, 0)),
            pl.BlockSpec((ROW_BLK, D_H), lambda i: (i, 0)),
        ],
        out_shape=[jax.ShapeDtypeStruct((n, D_H), jnp.float32)] * 2,
    )(x, w_l, w_r)


def _combine1_body(agg_ref, degp_ref, b_ref, r_ref, wl_ref, wr_ref,
                   p2_ref, r2_ref, rdeg_ref):
    deg = degp_ref[0][:, :1] + degp_ref[1][:, :1]
    rdeg = 1.0 / jnp.maximum(deg, 1.0)
    h = jnp.maximum(
        (agg_ref[0] + agg_ref[1]) * rdeg + b_ref[...] + r_ref[...], 0.0)
    p2_ref[...] = jnp.dot(h, wl_ref[...], preferred_element_type=jnp.float32)
    r2_ref[...] = jnp.dot(h, wr_ref[...], preferred_element_type=jnp.float32)
    rdeg_ref[...] = rdeg


def _combine1(agg, degp, b, r, w_l, w_r):
    return pl.pallas_call(
        _combine1_body,
        grid=(N_NODES // ROW_BLK,),
        in_specs=[
            pl.BlockSpec((NC, ROW_BLK, D_H), lambda i: (0, i, 0)),
            pl.BlockSpec((NC, ROW_BLK, 16), lambda i: (0, i, 0)),
            pl.BlockSpec((D_H,), lambda i: (0,)),
            pl.BlockSpec((ROW_BLK, D_H), lambda i: (i, 0)),
            pl.BlockSpec((D_H, D_H), lambda i: (0, 0)),
            pl.BlockSpec((D_H, D_H), lambda i: (0, 0)),
        ],
        out_specs=[
            pl.BlockSpec((ROW_BLK, D_H), lambda i: (i, 0)),
            pl.BlockSpec((ROW_BLK, D_H), lambda i: (i, 0)),
            pl.BlockSpec((ROW_BLK, 1), lambda i: (i, 0)),
        ],
        out_shape=[
            jax.ShapeDtypeStruct((N_NODES, D_H), jnp.float32),
            jax.ShapeDtypeStruct((N_NODES, D_H), jnp.float32),
            jax.ShapeDtypeStruct((N_NODES, 1), jnp.float32),
        ],
    )(agg, degp, b, r, w_l, w_r)


def _combine2_body(agg_ref, rdeg_ref, b_ref, r_ref, o_ref):
    o_ref[...] = jnp.maximum(
        (agg_ref[0] + agg_ref[1]) * rdeg_ref[...] + b_ref[...] + r_ref[...],
        0.0)


def _combine2(agg, rdeg, b, r):
    return pl.pallas_call(
        _combine2_body,
        grid=(N_NODES // ROW_BLK,),
        in_specs=[
            pl.BlockSpec((NC, ROW_BLK, D_H), lambda i: (0, i, 0)),
            pl.BlockSpec((ROW_BLK, 1), lambda i: (i, 0)),
            pl.BlockSpec((D_H,), lambda i: (0,)),
            pl.BlockSpec((ROW_BLK, D_H), lambda i: (i, 0)),
        ],
        out_specs=pl.BlockSpec((ROW_BLK, D_H), lambda i: (i, 0)),
        out_shape=jax.ShapeDtypeStruct((N_NODES, D_H), jnp.float32),
    )(agg, rdeg, b, r)


def _make_sc_aggregate(with_deg):
    mesh = plsc.VectorSubcoreMesh(core_axis_name="c", subcore_axis_name="s")
    out_type = [jax.ShapeDtypeStruct((NC, N_PAD, D_H), jnp.float32)]
    scratch = (
        [pltpu.VMEM((K,), jnp.int32) for _ in range(4)]
        + [pltpu.VMEM((K,), jnp.int32) for _ in range(4)]
        + [pltpu.VMEM((K, D_H), jnp.float32) for _ in range(2)]
        + [pltpu.VMEM_SHARED((N_PAD, D_H), jnp.float32)]
        + [pltpu.SemaphoreType.DMA] * 8
    )
    if with_deg:
        out_type.append(jax.ShapeDtypeStruct((NC, N_PAD, 16), jnp.float32))
        scratch += [
            pltpu.VMEM((N_PAD,), jnp.float32),
            pltpu.VMEM_SHARED((NS, N_PAD), jnp.float32),
            pltpu.VMEM((RPS + 16,), jnp.float32),
            pltpu.VMEM((RPS,), jnp.float32),
            pltpu.VMEM((RPS, 16), jnp.float32),
        ]

    def body(p_hbm, eflat_hbm, zeros_hbm, *rest):
        if with_deg:
            agg_out, deg_out = rest[0], rest[1]
            rest = rest[2:]
        else:
            agg_out = rest[0]
            rest = rest[1:]
        sidx = rest[0:4]
        didx = rest[4:8]
        rows = rest[8:10]
        aggsh = rest[10]
        sem_i = rest[11:15]
        sem_g = rest[15:17]
        sem_s = rest[17:19]
        if with_deg:
            ldeg, parts_sh, racc, rtmp, rspread = rest[19:24]
        c = lax.axis_index("c")
        s = lax.axis_index("s")
        wid = c * NS + s

        pltpu.sync_copy(zeros_hbm.at[pl.ds(s * RPS, RPS)],
                        aggsh.at[pl.ds(s * RPS, RPS)])
        if with_deg:
            z16 = jnp.zeros((16,), jnp.float32)

            def zero_deg(i, carry):
                ldeg[pl.ds(i * 16, 16)] = z16
                return carry
            lax.fori_loop(0, N_PAD // 16, zero_deg, 0)
        plsc.subcore_barrier()

        ones16 = jnp.ones((16,), jnp.float32)
        nj = CHUNKS + jnp.where(wid < EXTRA, 1, 0)

        def fire_idx(j, slot):
            off = (wid + j * NW) * K
            pltpu.async_copy(eflat_hbm.at[pl.ds(off, K)],
                             sidx[slot], sem_i[slot])
            pltpu.async_copy(eflat_hbm.at[pl.ds(N_EDGES + off, K)],
                             didx[slot], sem_i[slot])

        def wait_idx(j, slot):
            off = (wid + j * NW) * K
            pltpu.make_async_copy(eflat_hbm.at[pl.ds(off, K)],
                                  sidx[slot], sem_i[slot]).wait()
            pltpu.make_async_copy(eflat_hbm.at[pl.ds(N_EDGES + off, K)],
                                  didx[slot], sem_i[slot]).wait()

        fire_idx(0, 0)
        wait_idx(0, 0)
        fire_idx(1, 1)
        fire_idx(2, 2)
        pltpu.async_copy(p_hbm.at[sidx[0]], rows[0], sem_g[0])

        def chunk_step(j, b):
            r = b
            p = b % 2
            q = 1 - p
            rp = (b - 1) % 4
            rn = (b + 3) % 4

            @pl.when(j + 1 < nj)
            def _launch_next_gather():
                wait_idx(j + 1, (b + 1) % 4)

                @pl.when(j >= 1)
                def _drain_prev_scatter():
                    pltpu.make_async_copy(
                        rows[q], aggsh.at[didx[rp]], sem_s[q]).wait()
                pltpu.async_copy(
                    p_hbm.at[sidx[(b + 1) % 4]], rows[q], sem_g[q])

            pltpu.make_async_copy(
                p_hbm.at[sidx[r]], rows[p], sem_g[p]).wait()
            pltpu.async_copy(rows[p], aggsh.at[didx[r]], sem_s[p], add=True)
            if with_deg:
                for t in range(K // 16):
                    plsc.addupdate_scatter(
                        ldeg, [didx[r][pl.ds(t * 16, 16)]], ones16)

            @pl.when(j + 3 < nj)
            def _next_idx():
                fire_idx(j + 3, rn)

        def step(i, carry):
            j0 = i * 4
            for b in range(4):
                @pl.when(j0 + b < nj)
                def _do(jb=j0 + b, bb=b):
                    chunk_step(jb, bb)
            return carry
        lax.fori_loop(0, (CHUNKS + 1 + 3) // 4, step, 0)

        pltpu.make_async_copy(rows[0], aggsh.at[didx[0]], sem_s[0]).wait()
        pltpu.make_async_copy(rows[1], aggsh.at[didx[1]], sem_s[1]).wait()

        plsc.subcore_barrier()

        pltpu.sync_copy(aggsh.at[pl.ds(s * RPS, RPS)],
                        agg_out.at[c, pl.ds(s * RPS, RPS)])

        if with_deg:
            pltpu.sync_copy(ldeg, parts_sh.at[s])
            plsc.subcore_barrier()
            seg = s * RPS
            pltpu.sync_copy(parts_sh.at[0, pl.ds(seg, RPS)],
                            racc.at[pl.ds(0, RPS)])

            def red_k(k, carry):
                pltpu.sync_copy(parts_sh.at[k, pl.ds(seg, RPS)], rtmp)

                def red_v(t, carry2):
                    sl = pl.ds(t * 16, 16)
                    racc[sl] = racc[sl] + rtmp[sl]
                    return carry2
                lax.fori_loop(0, RPS // 16, red_v, 0)
                return carry
            lax.fori_loop(1, NS, red_k, 0)

            def spread(i, carry):
                rspread[i, pl.ds(0, 16)] = racc[pl.ds(i, 16)]
                return carry
            lax.fori_loop(0, RPS, spread, 0)
            pltpu.sync_copy(rspread, deg_out.at[c, pl.ds(seg, RPS)])

    return pl.kernel(
        body,
        out_type=tuple(out_type) if with_deg else out_type[0],
        mesh=mesh,
        scratch_types=scratch,
        compiler_params=pltpu.CompilerParams(use_tc_tiling_on_sc=False,
                                             needs_layout_passes=False),
    )


_sc_aggregate_deg = _make_sc_aggregate(with_deg=True)
_sc_aggregate = _make_sc_aggregate(with_deg=False)


def kernel(x, edge_index, W1_l, b1, W1_r, W2_l, b2, W2_r):
    eflat = edge_index.astype(jnp.int32).reshape(2 * N_EDGES)
    zeros = jnp.zeros((N_PAD, D_H), jnp.float32)

    p1, r1 = _project(x, W1_l, W1_r)
    agg1, degp = _sc_aggregate_deg(p1, eflat, zeros)
    p2, r2, rdeg = _combine1(agg1, degp, b1, r1, W2_l, W2_r)
    agg2 = _sc_aggregate(p2, eflat, zeros)
    return _combine2(agg2, rdeg, b2, r2)


# speedup vs baseline: 19.2665x; 1.0288x over previous
import functools

import jax
import jax.numpy as jnp
from jax import lax
from jax.experimental import pallas as pl
from jax.experimental.pallas import tpu as pltpu
from jax.experimental.pallas import tpu_sc as plsc

N_NODES = 10000
N_EDGES = 320000
D_IN = 128
D_H = 32

NC, NS = 2, 16
NW = NC * NS
K = 128
CHUNKS = N_EDGES // (NW * K)
EXTRA = (N_EDGES // K) % NW
N_PAD = 10240
RPS = N_PAD // NS
RB = 3
IR = 6

ROW_BLK = 2000


def _project_body(x_ref, wl_ref, wr_ref, p_ref, r_ref):
    xb = x_ref[...]
    p_ref[...] = jnp.dot(xb, wl_ref[...], preferred_element_type=jnp.float32)
    r_ref[...] = jnp.dot(xb, wr_ref[...], preferred_element_type=jnp.float32)


def _project(x, w_l, w_r):
    n, d = x.shape
    return pl.pallas_call(
        _project_body,
        grid=(n // ROW_BLK,),
        in_specs=[
            pl.BlockSpec((ROW_BLK, d), lambda i: (i, 0)),
            pl.BlockSpec((d, D_H), lambda i: (0, 0)),
            pl.BlockSpec((d, D_H), lambda i: (0, 0)),
        ],
        out_specs=[
            pl.BlockSpec((ROW_BLK, D_H), lambda i: (i, 0)),
            pl.BlockSpec((ROW_BLK, D_H), lambda i: (i, 0)),
        ],
        out_shape=[jax.ShapeDtypeStruct((n, D_H), jnp.float32)] * 2,
    )(x, w_l, w_r)


def _combine1_body(agg_ref, degp_ref, b_ref, r_ref, wl_ref, wr_ref,
                   p2_ref, r2_ref, rdeg_ref):
    deg = degp_ref[0][:, :1] + degp_ref[1][:, :1]
    rdeg = 1.0 / jnp.maximum(deg, 1.0)
    h = jnp.maximum(
        (agg_ref[0] + agg_ref[1]) * rdeg + b_ref[...] + r_ref[...], 0.0)
    p2_ref[...] = jnp.dot(h, wl_ref[...], preferred_element_type=jnp.float32)
    r2_ref[...] = jnp.dot(h, wr_ref[...], preferred_element_type=jnp.float32)
    rdeg_ref[...] = rdeg


def _combine1(agg, degp, b, r, w_l, w_r):
    return pl.pallas_call(
        _combine1_body,
        grid=(N_NODES // ROW_BLK,),
        in_specs=[
            pl.BlockSpec((NC, ROW_BLK, D_H), lambda i: (0, i, 0)),
            pl.BlockSpec((NC, ROW_BLK, 16), lambda i: (0, i, 0)),
            pl.BlockSpec((D_H,), lambda i: (0,)),
            pl.BlockSpec((ROW_BLK, D_H), lambda i: (i, 0)),
            pl.BlockSpec((D_H, D_H), lambda i: (0, 0)),
            pl.BlockSpec((D_H, D_H), lambda i: (0, 0)),
        ],
        out_specs=[
            pl.BlockSpec((ROW_BLK, D_H), lambda i: (i, 0)),
            pl.BlockSpec((ROW_BLK, D_H), lambda i: (i, 0)),
            pl.BlockSpec((ROW_BLK, 1), lambda i: (i, 0)),
        ],
        out_shape=[
            jax.ShapeDtypeStruct((N_NODES, D_H), jnp.float32),
            jax.ShapeDtypeStruct((N_NODES, D_H), jnp.float32),
            jax.ShapeDtypeStruct((N_NODES, 1), jnp.float32),
        ],
    )(agg, degp, b, r, w_l, w_r)


def _combine2_body(agg_ref, rdeg_ref, b_ref, r_ref, o_ref):
    o_ref[...] = jnp.maximum(
        (agg_ref[0] + agg_ref[1]) * rdeg_ref[...] + b_ref[...] + r_ref[...],
        0.0)


def _combine2(agg, rdeg, b, r):
    return pl.pallas_call(
        _combine2_body,
        grid=(N_NODES // ROW_BLK,),
        in_specs=[
            pl.BlockSpec((NC, ROW_BLK, D_H), lambda i: (0, i, 0)),
            pl.BlockSpec((ROW_BLK, 1), lambda i: (i, 0)),
            pl.BlockSpec((D_H,), lambda i: (0,)),
            pl.BlockSpec((ROW_BLK, D_H), lambda i: (i, 0)),
        ],
        out_specs=pl.BlockSpec((ROW_BLK, D_H), lambda i: (i, 0)),
        out_shape=jax.ShapeDtypeStruct((N_NODES, D_H), jnp.float32),
    )(agg, rdeg, b, r)


def _make_sc_aggregate(with_deg):
    mesh = plsc.VectorSubcoreMesh(core_axis_name="c", subcore_axis_name="s")
    out_type = [jax.ShapeDtypeStruct((NC, N_PAD, D_H), jnp.float32)]
    scratch = (
        [pltpu.VMEM((K,), jnp.int32) for _ in range(IR)]
        + [pltpu.VMEM((K,), jnp.int32) for _ in range(IR)]
        + [pltpu.VMEM((K, D_H), jnp.float32) for _ in range(RB)]
        + [pltpu.VMEM_SHARED((N_PAD, D_H), jnp.float32)]
        + [pltpu.SemaphoreType.DMA] * (IR + 2 * RB)
    )
    if with_deg:
        out_type.append(jax.ShapeDtypeStruct((NC, N_PAD, 16), jnp.float32))
        scratch += [
            pltpu.VMEM((N_PAD,), jnp.float32),
            pltpu.VMEM_SHARED((NS, N_PAD), jnp.float32),
            pltpu.VMEM((RPS + 16,), jnp.float32),
            pltpu.VMEM((RPS,), jnp.float32),
            pltpu.VMEM((RPS, 16), jnp.float32),
        ]

    def body(p_hbm, eflat_hbm, zeros_hbm, *rest):
        if with_deg:
            agg_out, deg_out = rest[0], rest[1]
            rest = rest[2:]
        else:
            agg_out = rest[0]
            rest = rest[1:]
        sidx = rest[0:IR]
        didx = rest[IR:2 * IR]
        rows = rest[2 * IR:2 * IR + RB]
        aggsh = rest[2 * IR + RB]
        o = 2 * IR + RB + 1
        sem_i = rest[o:o + IR]
        sem_g = rest[o + IR:o + IR + RB]
        sem_s = rest[o + IR + RB:o + IR + 2 * RB]
        if with_deg:
            ldeg, parts_sh, racc, rtmp, rspread = rest[o + IR + 2 * RB:]
        c = lax.axis_index("c")
        s = lax.axis_index("s")
        wid = c * NS + s

        pltpu.sync_copy(zeros_hbm.at[pl.ds(s * RPS, RPS)],
                        aggsh.at[pl.ds(s * RPS, RPS)])
        if with_deg:
            z16 = jnp.zeros((16,), jnp.float32)

            def zero_deg(i, carry):
                ldeg[pl.ds(i * 16, 16)] = z16
                return carry
            lax.fori_loop(0, N_PAD // 16, zero_deg, 0)
        plsc.subcore_barrier()

        ones16 = jnp.ones((16,), jnp.float32)
        nj = CHUNKS + jnp.where(wid < EXTRA, 1, 0)

        def fire_idx(j, slot):
            off = (wid + j * NW) * K
            pltpu.async_copy(eflat_hbm.at[pl.ds(off, K)],
                             sidx[slot], sem_i[slot])
            pltpu.async_copy(eflat_hbm.at[pl.ds(N_EDGES + off, K)],
                             didx[slot], sem_i[slot])

        def wait_idx(j, slot):
            off = (wid + j * NW) * K
            pltpu.make_async_copy(eflat_hbm.at[pl.ds(off, K)],
                                  sidx[slot], sem_i[slot]).wait()
            pltpu.make_async_copy(eflat_hbm.at[pl.ds(N_EDGES + off, K)],
                                  didx[slot], sem_i[slot]).wait()

        fire_idx(0, 0)
        wait_idx(0, 0)
        for jj in range(1, IR // 2):
            fire_idx(jj, jj)
        pltpu.async_copy(p_hbm.at[sidx[0]], rows[0], sem_g[0])

        def chunk_step(j, b):
            r = b % IR
            p = b % RB

            @pl.when(j + 1 < nj)
            def _launch_next_gather():
                wait_idx(j + 1, (b + 1) % IR)

                @pl.when(j + 1 >= RB)
                def _drain_old_scatter():
                    pltpu.make_async_copy(
                        rows[(b + 1) % RB], aggsh.at[didx[(b + 1) % IR]],
                        sem_s[(b + 1) % RB]).wait()
                pltpu.async_copy(
                    p_hbm.at[sidx[(b + 1) % IR]], rows[(b + 1) % RB],
                    sem_g[(b + 1) % RB])

            pltpu.make_async_copy(
                p_hbm.at[sidx[r]], rows[p], sem_g[p]).wait()
            pltpu.async_copy(rows[p], aggsh.at[didx[r]], sem_s[p], add=True)
            if with_deg:
                for t in range(K // 16):
                    plsc.addupdate_scatter(
                        ldeg, [didx[r][pl.ds(t * 16, 16)]], ones16)

            @pl.when(j + IR // 2 < nj)
            def _next_idx():
                fire_idx(j + IR // 2, (b + IR // 2) % IR)

        def step(i, carry):
            j0 = i * IR
            for b in range(IR):
                @pl.when(j0 + b < nj)
                def _do(jb=j0 + b, bb=b):
                    chunk_step(jb, bb)
            return carry
        lax.fori_loop(0, (CHUNKS + 1 + IR - 1) // IR, step, 0)

        for p in range(RB):
            pltpu.make_async_copy(rows[p], aggsh.at[didx[p]],
                                  sem_s[p]).wait()

        plsc.subcore_barrier()

        pltpu.sync_copy(aggsh.at[pl.ds(s * RPS, RPS)],
                        agg_out.at[c, pl.ds(s * RPS, RPS)])

        if with_deg:
            pltpu.sync_copy(ldeg, parts_sh.at[s])
            plsc.subcore_barrier()
            seg = s * RPS
            pltpu.sync_copy(parts_sh.at[0, pl.ds(seg, RPS)],
                            racc.at[pl.ds(0, RPS)])

            def red_k(k, carry):
                pltpu.sync_copy(parts_sh.at[k, pl.ds(seg, RPS)], rtmp)

                def red_v(t, carry2):
                    sl = pl.ds(t * 16, 16)
                    racc[sl] = racc[sl] + rtmp[sl]
                    return carry2
                lax.fori_loop(0, RPS // 16, red_v, 0)
                return carry
            lax.fori_loop(1, NS, red_k, 0)

            def spread(i, carry):
                rspread[i, pl.ds(0, 16)] = racc[pl.ds(i, 16)]
                return carry
            lax.fori_loop(0, RPS, spread, 0)
            pltpu.sync_copy(rspread, deg_out.at[c, pl.ds(seg, RPS)])

    return pl.kernel(
        body,
        out_type=tuple(out_type) if with_deg else out_type[0],
        mesh=mesh,
        scratch_types=scratch,
        compiler_params=pltpu.CompilerParams(use_tc_tiling_on_sc=False,
                                             needs_layout_passes=False),
    )


_sc_aggregate_deg = _make_sc_aggregate(with_deg=True)
_sc_aggregate = _make_sc_aggregate(with_deg=False)


def kernel(x, edge_index, W1_l, b1, W1_r, W2_l, b2, W2_r):
    eflat = edge_index.astype(jnp.int32).reshape(2 * N_EDGES)
    zeros = jnp.zeros((N_PAD, D_H), jnp.float32)

    p1, r1 = _project(x, W1_l, W1_r)
    agg1, degp = _sc_aggregate_deg(p1, eflat, zeros)
    p2, r2, rdeg = _combine1(agg1, degp, b1, r1, W2_l, W2_r)
    agg2 = _sc_aggregate(p2, eflat, zeros)
    return _combine2(agg2, rdeg, b2, r2)


# speedup vs baseline: 19.6100x; 1.0178x over previous
import functools

import jax
import jax.numpy as jnp
from jax import lax
from jax.experimental import pallas as pl
from jax.experimental.pallas import tpu as pltpu
from jax.experimental.pallas import tpu_sc as plsc

N_NODES = 10000
N_EDGES = 320000
D_IN = 128
D_H = 32

NC, NS = 2, 16
NW = NC * NS
K = 128
CHUNKS = N_EDGES // (NW * K)
EXTRA = (N_EDGES // K) % NW
N_PAD = 10240
RPS = N_PAD // NS
RB = 4
IR = 8

ROW_BLK = 2000


def _project_body(x_ref, wl_ref, wr_ref, p_ref, r_ref):
    xb = x_ref[...]
    p_ref[...] = jnp.dot(xb, wl_ref[...], preferred_element_type=jnp.float32)
    r_ref[...] = jnp.dot(xb, wr_ref[...], preferred_element_type=jnp.float32)


def _project(x, w_l, w_r):
    n, d = x.shape
    return pl.pallas_call(
        _project_body,
        grid=(n // ROW_BLK,),
        in_specs=[
            pl.BlockSpec((ROW_BLK, d), lambda i: (i, 0)),
            pl.BlockSpec((d, D_H), lambda i: (0, 0)),
            pl.BlockSpec((d, D_H), lambda i: (0, 0)),
        ],
        out_specs=[
            pl.BlockSpec((ROW_BLK, D_H), lambda i: (i, 0)),
            pl.BlockSpec((ROW_BLK, D_H), lambda i: (i, 0)),
        ],
        out_shape=[jax.ShapeDtypeStruct((n, D_H), jnp.float32)] * 2,
    )(x, w_l, w_r)


def _combine1_body(agg_ref, degp_ref, b_ref, r_ref, wl_ref, wr_ref,
                   p2_ref, r2_ref, rdeg_ref):
    deg = degp_ref[0][:, :1] + degp_ref[1][:, :1]
    rdeg = 1.0 / jnp.maximum(deg, 1.0)
    h = jnp.maximum(
        (agg_ref[0] + agg_ref[1]) * rdeg + b_ref[...] + r_ref[...], 0.0)
    p2_ref[...] = jnp.dot(h, wl_ref[...], preferred_element_type=jnp.float32)
    r2_ref[...] = jnp.dot(h, wr_ref[...], preferred_element_type=jnp.float32)
    rdeg_ref[...] = rdeg


def _combine1(agg, degp, b, r, w_l, w_r):
    return pl.pallas_call(
        _combine1_body,
        grid=(N_NODES // ROW_BLK,),
        in_specs=[
            pl.BlockSpec((NC, ROW_BLK, D_H), lambda i: (0, i, 0)),
            pl.BlockSpec((NC, ROW_BLK, 16), lambda i: (0, i, 0)),
            pl.BlockSpec((D_H,), lambda i: (0,)),
            pl.BlockSpec((ROW_BLK, D_H), lambda i: (i, 0)),
            pl.BlockSpec((D_H, D_H), lambda i: (0, 0)),
            pl.BlockSpec((D_H, D_H), lambda i: (0, 0)),
        ],
        out_specs=[
            pl.BlockSpec((ROW_BLK, D_H), lambda i: (i, 0)),
            pl.BlockSpec((ROW_BLK, D_H), lambda i: (i, 0)),
            pl.BlockSpec((ROW_BLK, 1), lambda i: (i, 0)),
        ],
        out_shape=[
            jax.ShapeDtypeStruct((N_NODES, D_H), jnp.float32),
            jax.ShapeDtypeStruct((N_NODES, D_H), jnp.float32),
            jax.ShapeDtypeStruct((N_NODES, 1), jnp.float32),
        ],
    )(agg, degp, b, r, w_l, w_r)


def _combine2_body(agg_ref, rdeg_ref, b_ref, r_ref, o_ref):
    o_ref[...] = jnp.maximum(
        (agg_ref[0] + agg_ref[1]) * rdeg_ref[...] + b_ref[...] + r_ref[...],
        0.0)


def _combine2(agg, rdeg, b, r):
    return pl.pallas_call(
        _combine2_body,
        grid=(N_NODES // ROW_BLK,),
        in_specs=[
            pl.BlockSpec((NC, ROW_BLK, D_H), lambda i: (0, i, 0)),
            pl.BlockSpec((ROW_BLK, 1), lambda i: (i, 0)),
            pl.BlockSpec((D_H,), lambda i: (0,)),
            pl.BlockSpec((ROW_BLK, D_H), lambda i: (i, 0)),
        ],
        out_specs=pl.BlockSpec((ROW_BLK, D_H), lambda i: (i, 0)),
        out_shape=jax.ShapeDtypeStruct((N_NODES, D_H), jnp.float32),
    )(agg, rdeg, b, r)


def _make_sc_aggregate(with_deg):
    mesh = plsc.VectorSubcoreMesh(core_axis_name="c", subcore_axis_name="s")
    out_type = [jax.ShapeDtypeStruct((NC, N_PAD, D_H), jnp.float32)]
    scratch = (
        [pltpu.VMEM((K,), jnp.int32) for _ in range(IR)]
        + [pltpu.VMEM((K,), jnp.int32) for _ in range(IR)]
        + [pltpu.VMEM((K, D_H), jnp.float32) for _ in range(RB)]
        + [pltpu.VMEM_SHARED((N_PAD, D_H), jnp.float32)]
        + [pltpu.SemaphoreType.DMA] * (IR + 2 * RB)
    )
    if with_deg:
        out_type.append(jax.ShapeDtypeStruct((NC, N_PAD, 16), jnp.float32))
        scratch += [
            pltpu.VMEM((N_PAD,), jnp.float32),
            pltpu.VMEM_SHARED((NS, N_PAD), jnp.float32),
            pltpu.VMEM((RPS + 16,), jnp.float32),
            pltpu.VMEM((RPS,), jnp.float32),
            pltpu.VMEM((RPS, 16), jnp.float32),
        ]

    def body(p_hbm, eflat_hbm, zeros_hbm, *rest):
        if with_deg:
            agg_out, deg_out = rest[0], rest[1]
            rest = rest[2:]
        else:
            agg_out = rest[0]
            rest = rest[1:]
        sidx = rest[0:IR]
        didx = rest[IR:2 * IR]
        rows = rest[2 * IR:2 * IR + RB]
        aggsh = rest[2 * IR + RB]
        o = 2 * IR + RB + 1
        sem_i = rest[o:o + IR]
        sem_g = rest[o + IR:o + IR + RB]
        sem_s = rest[o + IR + RB:o + IR + 2 * RB]
        if with_deg:
            ldeg, parts_sh, racc, rtmp, rspread = rest[o + IR + 2 * RB:]
        c = lax.axis_index("c")
        s = lax.axis_index("s")
        wid = c * NS + s

        pltpu.sync_copy(zeros_hbm.at[pl.ds(s * RPS, RPS)],
                        aggsh.at[pl.ds(s * RPS, RPS)])
        if with_deg:
            z16 = jnp.zeros((16,), jnp.float32)

            def zero_deg(i, carry):
                ldeg[pl.ds(i * 16, 16)] = z16
                return carry
            lax.fori_loop(0, N_PAD // 16, zero_deg, 0)
        plsc.subcore_barrier()

        ones16 = jnp.ones((16,), jnp.float32)
        nj = CHUNKS + jnp.where(wid < EXTRA, 1, 0)

        def fire_idx(j, slot):
            off = (wid + j * NW) * K
            pltpu.async_copy(eflat_hbm.at[pl.ds(off, K)],
                             sidx[slot], sem_i[slot])
            pltpu.async_copy(eflat_hbm.at[pl.ds(N_EDGES + off, K)],
                             didx[slot], sem_i[slot])

        def wait_idx(j, slot):
            off = (wid + j * NW) * K
            pltpu.make_async_copy(eflat_hbm.at[pl.ds(off, K)],
                                  sidx[slot], sem_i[slot]).wait()
            pltpu.make_async_copy(eflat_hbm.at[pl.ds(N_EDGES + off, K)],
                                  didx[slot], sem_i[slot]).wait()

        fire_idx(0, 0)
        wait_idx(0, 0)
        for jj in range(1, IR // 2):
            fire_idx(jj, jj)
        pltpu.async_copy(p_hbm.at[sidx[0]], rows[0], sem_g[0])

        def chunk_step(j, b):
            r = b % IR
            p = b % RB

            @pl.when(j + 1 < nj)
            def _launch_next_gather():
                wait_idx(j + 1, (b + 1) % IR)

                @pl.when(j + 1 >= RB)
                def _drain_old_scatter():
                    pltpu.make_async_copy(
                        rows[(b + 1) % RB], aggsh.at[didx[(b + 1) % IR]],
                        sem_s[(b + 1) % RB]).wait()
                pltpu.async_copy(
                    p_hbm.at[sidx[(b + 1) % IR]], rows[(b + 1) % RB],
                    sem_g[(b + 1) % RB])

            pltpu.make_async_copy(
                p_hbm.at[sidx[r]], rows[p], sem_g[p]).wait()
            pltpu.async_copy(rows[p], aggsh.at[didx[r]], sem_s[p], add=True)
            if with_deg:
                for t in range(K // 16):
                    plsc.addupdate_scatter(
                        ldeg, [didx[r][pl.ds(t * 16, 16)]], ones16)

            @pl.when(j + IR // 2 < nj)
            def _next_idx():
                fire_idx(j + IR // 2, (b + IR // 2) % IR)

        def step(i, carry):
            j0 = i * IR
            for b in range(IR):
                @pl.when(j0 + b < nj)
                def _do(jb=j0 + b, bb=b):
                    chunk_step(jb, bb)
            return carry
        lax.fori_loop(0, (CHUNKS + 1 + IR - 1) // IR, step, 0)

        for p in range(RB):
            pltpu.make_async_copy(rows[p], aggsh.at[didx[p]],
                                  sem_s[p]).wait()

        plsc.subcore_barrier()

        pltpu.sync_copy(aggsh.at[pl.ds(s * RPS, RPS)],
                        agg_out.at[c, pl.ds(s * RPS, RPS)])

        if with_deg:
            pltpu.sync_copy(ldeg, parts_sh.at[s])
            plsc.subcore_barrier()
            seg = s * RPS
            pltpu.sync_copy(parts_sh.at[0, pl.ds(seg, RPS)],
                            racc.at[pl.ds(0, RPS)])

            def red_k(k, carry):
                pltpu.sync_copy(parts_sh.at[k, pl.ds(seg, RPS)], rtmp)

                def red_v(t, carry2):
                    sl = pl.ds(t * 16, 16)
                    racc[sl] = racc[sl] + rtmp[sl]
                    return carry2
                lax.fori_loop(0, RPS // 16, red_v, 0)
                return carry
            lax.fori_loop(1, NS, red_k, 0)

            def spread(i, carry):
                rspread[i, pl.ds(0, 16)] = racc[pl.ds(i, 16)]
                return carry
            lax.fori_loop(0, RPS, spread, 0)
            pltpu.sync_copy(rspread, deg_out.at[c, pl.ds(seg, RPS)])

    return pl.kernel(
        body,
        out_type=tuple(out_type) if with_deg else out_type[0],
        mesh=mesh,
        scratch_types=scratch,
        compiler_params=pltpu.CompilerParams(use_tc_tiling_on_sc=False,
                                             needs_layout_passes=False),
    )


_sc_aggregate_deg = _make_sc_aggregate(with_deg=True)
_sc_aggregate = _make_sc_aggregate(with_deg=False)


def kernel(x, edge_index, W1_l, b1, W1_r, W2_l, b2, W2_r):
    eflat = edge_index.astype(jnp.int32).reshape(2 * N_EDGES)
    zeros = jnp.zeros((N_PAD, D_H), jnp.float32)

    p1, r1 = _project(x, W1_l, W1_r)
    agg1, degp = _sc_aggregate_deg(p1, eflat, zeros)
    p2, r2, rdeg = _combine1(agg1, degp, b1, r1, W2_l, W2_r)
    agg2 = _sc_aggregate(p2, eflat, zeros)
    return _combine2(agg2, rdeg, b2, r2)
